# baseline (speedup 1.0000x reference)
import jax
import jax.numpy as jnp
from jax import lax
from jax.experimental import pallas as pl
from jax.experimental.pallas import tpu as pltpu
from jax.experimental.pallas import tpu_sc as plsc

VOCAB = 1000000
EMBED = 64
MAX_SEQ = 200
BATCH = 4096

NUM_CORES = 2
NUM_SUBCORES = 16
NUM_WORKERS = NUM_CORES * NUM_SUBCORES
ROWS_PER_WORKER = BATCH // NUM_WORKERS
PAIR = 2
PAIRS_PER_WORKER = ROWS_PER_WORKER // PAIR
IDX_SPLITS = ((0, 128), (128, 72))
LANES = 16


def _body(x_hbm, we_hbm, out_hbm, idx_v, rows_v, gsem, ssem):
    wid = lax.axis_index("s") * NUM_CORES + lax.axis_index("c")
    row_base = wid * ROWS_PER_WORKER

    pltpu.sync_copy(x_hbm.at[pl.ds(row_base, ROWS_PER_WORKER)], idx_v)

    def fire_gathers(slot, p):
        for k in range(PAIR):
            for off, ln in IDX_SPLITS:
                pltpu.async_copy(
                    we_hbm.at[idx_v.at[PAIR * p + k, pl.ds(off, ln)]],
                    rows_v.at[slot, k, pl.ds(off, ln)],
                    gsem.at[slot],
                )

    def wait_gathers(slot):
        for k in range(PAIR):
            pltpu.make_async_copy(
                we_hbm.at[pl.ds(0, MAX_SEQ)], rows_v.at[slot, k], gsem.at[slot]
            ).wait()

    def fire_store(slot, p):
        pltpu.async_copy(
            rows_v.at[slot],
            out_hbm.at[pl.ds(row_base + PAIR * p, PAIR)],
            ssem.at[slot],
        )

    def wait_store(slot):
        pltpu.make_async_copy(
            rows_v.at[slot], out_hbm.at[pl.ds(0, PAIR)], ssem.at[slot]
        ).wait()

    fire_gathers(0, 0)

    @pl.loop(0, PAIRS_PER_WORKER, step=2)
    def _(p0):
        for q in range(2):
            p = p0 + q
            s = q
            o = 1 - q

            @pl.when(p >= 1)
            def _():
                wait_store(o)

            @pl.when(p + 1 < PAIRS_PER_WORKER)
            def _():
                fire_gathers(o, p + 1)

            wait_gathers(s)
            fire_store(s, p)

    wait_store(1)


def _sc_gather(x, we_table):
    mesh = plsc.VectorSubcoreMesh(
        core_axis_name="c", subcore_axis_name="s", num_cores=NUM_CORES,
        num_subcores=NUM_SUBCORES,
    )
    return pl.kernel(
        _body,
        out_type=jax.ShapeDtypeStruct((BATCH, MAX_SEQ, EMBED), jnp.float32),
        mesh=mesh,
        scratch_types=[
            pltpu.VMEM((ROWS_PER_WORKER, MAX_SEQ), jnp.int32),
            pltpu.VMEM((2, PAIR, MAX_SEQ, EMBED), jnp.float32),
            pltpu.SemaphoreType.DMA((2,)),
            pltpu.SemaphoreType.DMA((2,)),
        ],
        compiler_params=pltpu.CompilerParams(use_tc_tiling_on_sc=False),
    )(x, we_table)


BTILE = 512
STILE = 8


def _tc_body(rows_ref, pe_ref, ident_ref, out_ref):
    rows = rows_ref[...].reshape(BTILE, STILE * EMBED)
    outt = jax.lax.dot_general(
        rows, ident_ref[...], (((0,), (0,)), ((), ())),
        preferred_element_type=jnp.float32,
    )
    out = outt.reshape(STILE, EMBED, BTILE)
    out_ref[...] = out + pe_ref[...][:, :, None]


def _tc_fixup(scratch, pe_table):
    ident = jnp.eye(BTILE, dtype=jnp.float32)
    grid = (MAX_SEQ // STILE, BATCH // BTILE)
    return pl.pallas_call(
        _tc_body,
        grid=grid,
        in_specs=[
            pl.BlockSpec((BTILE, STILE, EMBED), lambda s, b: (b, s, 0)),
            pl.BlockSpec((STILE, EMBED), lambda s, b: (s, 0)),
            pl.BlockSpec((BTILE, BTILE), lambda s, b: (0, 0)),
        ],
        out_specs=pl.BlockSpec((STILE, EMBED, BTILE), lambda s, b: (s, 0, b)),
        out_shape=jax.ShapeDtypeStruct((MAX_SEQ, EMBED, BATCH), jnp.float32),
    )(scratch, pe_table, ident)


@jax.jit
def _emb_kernel(x, we_table, pe_table):
    scratch = _sc_gather(x, we_table)
    outT = _tc_fixup(scratch, pe_table)
    return outT.transpose(2, 0, 1)


def kernel(x, we_table, pe_table):
    return _emb_kernel(x, we_table, pe_table)
